# Initial kernel scaffold; baseline (speedup 1.0000x reference)
import jax
import jax.numpy as jnp
from jax.experimental import pallas as pl


def kernel(x, edge_index, edge_attr7, dist_sigma, coexpr_log1p, coexpr_scale, temperature, Wq, Wk, Wv, lr_table, phi_W1, phi_b1, phi_W2, phi_b2, wb, proj_W, proj_b):
    raise NotImplementedError("write your pallas kernel here")


# speedup vs baseline: 1.1186x; 1.1186x over previous
import functools
import math

import jax
import jax.numpy as jnp
import numpy as np
from jax.experimental import pallas as pl
from jax.experimental.pallas import tpu as pltpu

N = 10000
E = 160000
IN_DIM = 256
OUT_DIM = 256
HEADS = 8
DK = OUT_DIM // HEADS
LR_VOCAB = 512
LR_DIM = 32
EDGE_HID = 64
EDGE_BIAS = 32
FIJ_DIM = 6 + LR_DIM


def _qkv_body(x_ref, w_ref, out_ref):
    out_ref[...] = jnp.dot(x_ref[...], w_ref[...],
                           preferred_element_type=jnp.float32)


def _qkv(x, w_all):
    return pl.pallas_call(
        _qkv_body,
        out_shape=jax.ShapeDtypeStruct((N, 3 * OUT_DIM), jnp.float32),
        grid=(5,),
        in_specs=[
            pl.BlockSpec((N // 5, IN_DIM), lambda i: (i, 0)),
            pl.BlockSpec((IN_DIM, 3 * OUT_DIM), lambda i: (0, 0)),
        ],
        out_specs=pl.BlockSpec((N // 5, 3 * OUT_DIM), lambda i: (i, 0)),
    )(x, w_all)


EB = 8000


def _edge_body(cont_ref, lrid_ref, dot_ref, lrt_ref, w1_ref, b1_ref, w2_ref,
               b2_ref, wb_ref, eraw_ref):
    cont = cont_ref[...]
    lrid = lrid_ref[...]
    onehot = (lrid == jax.lax.broadcasted_iota(jnp.int32, (EB, LR_VOCAB), 1)
              ).astype(jnp.float32)
    lr_vec = jnp.dot(onehot, lrt_ref[...], preferred_element_type=jnp.float32)
    fij = jnp.concatenate([cont[:, :6], lr_vec], axis=1)
    h1 = jnp.maximum(jnp.dot(fij, w1_ref[...],
                             preferred_element_type=jnp.float32) + b1_ref[...], 0.0)
    phi = jnp.maximum(jnp.dot(h1, w2_ref[...],
                              preferred_element_type=jnp.float32) + b2_ref[...], 0.0)
    bias = jnp.dot(phi, wb_ref[...], preferred_element_type=jnp.float32)
    eraw_ref[...] = dot_ref[...] + bias


def _edge_bias(cont, lrid, dot, lr_table, phi_W1, phi_b1, phi_W2, phi_b2, wb):
    grid = (E // EB,)
    return pl.pallas_call(
        _edge_body,
        out_shape=jax.ShapeDtypeStruct((E, HEADS), jnp.float32),
        grid=grid,
        in_specs=[
            pl.BlockSpec((EB, 8), lambda i: (i, 0)),
            pl.BlockSpec((EB, 1), lambda i: (i, 0)),
            pl.BlockSpec((EB, HEADS), lambda i: (i, 0)),
            pl.BlockSpec((LR_VOCAB, LR_DIM), lambda i: (0, 0)),
            pl.BlockSpec((FIJ_DIM, EDGE_HID), lambda i: (0, 0)),
            pl.BlockSpec((1, EDGE_HID), lambda i: (0, 0)),
            pl.BlockSpec((EDGE_HID, EDGE_BIAS), lambda i: (0, 0)),
            pl.BlockSpec((1, EDGE_BIAS), lambda i: (0, 0)),
            pl.BlockSpec((EDGE_BIAS, HEADS), lambda i: (0, 0)),
        ],
        out_specs=pl.BlockSpec((EB, HEADS), lambda i: (i, 0)),
    )(cont, lrid, dot, lr_table, phi_W1, phi_b1, phi_W2, phi_b2, wb)


def _proj_body(out_ref, w_ref, b_ref, res_ref):
    res_ref[...] = (jnp.dot(out_ref[...], w_ref[...],
                            preferred_element_type=jnp.float32) + b_ref[...])


def _final_proj(out, proj_W, proj_b):
    return pl.pallas_call(
        _proj_body,
        out_shape=jax.ShapeDtypeStruct((N, OUT_DIM), jnp.float32),
        grid=(5,),
        in_specs=[
            pl.BlockSpec((N // 5, OUT_DIM), lambda i: (i, 0)),
            pl.BlockSpec((OUT_DIM, OUT_DIM), lambda i: (0, 0)),
            pl.BlockSpec((1, OUT_DIM), lambda i: (0, 0)),
        ],
        out_specs=pl.BlockSpec((N // 5, OUT_DIM), lambda i: (i, 0)),
    )(out, proj_W, proj_b)


def kernel(x, edge_index, edge_attr7, dist_sigma, coexpr_log1p, coexpr_scale,
           temperature, Wq, Wk, Wv, lr_table, phi_W1, phi_b1, phi_W2, phi_b2,
           wb, proj_W, proj_b):
    src = edge_index[0]
    dst = edge_index[1]

    w_all = jnp.concatenate([Wq.T, Wk.T, Wv.T], axis=1)
    qkv = _qkv(x, w_all)
    q = qkv[:, :OUT_DIM].reshape(N, HEADS, DK)
    k = qkv[:, OUT_DIM:2 * OUT_DIM].reshape(N, HEADS, DK)
    v = qkv[:, 2 * OUT_DIM:].reshape(N, HEADS, DK)

    qj = q[dst]
    ki = k[src]
    vi = v[src]
    tau = jnp.maximum(jnp.asarray(temperature, jnp.float32), 1e-6)
    dot = (qj * ki).sum(axis=-1) / np.sqrt(DK) / tau

    dist = edge_attr7[:, 0]
    coexpr = edge_attr7[:, 1]
    lr_id = edge_attr7[:, 2].astype(jnp.int32)
    cmi4 = edge_attr7[:, 3:7]
    sigma = jnp.maximum(jnp.asarray(dist_sigma, jnp.float32), 1e-6)
    dist_decay = jnp.exp(-dist / sigma)
    scale = jnp.maximum(jnp.asarray(coexpr_scale, jnp.float32), 1e-6)
    coexpr_norm = jnp.where(coexpr_log1p,
                            jnp.log1p(jnp.maximum(coexpr, 0.0)) / scale,
                            coexpr / scale)
    cmi4_norm = jnp.clip(cmi4, 0.0, 1.0)
    cont = jnp.concatenate([dist_decay[:, None], coexpr_norm[:, None],
                            cmi4_norm, jnp.zeros((E, 2), jnp.float32)], axis=1)

    e_raw = _edge_bias(cont, lr_id[:, None], dot, lr_table,
                       phi_W1.T, phi_b1[None, :], phi_W2.T, phi_b2[None, :],
                       wb.T)

    cnt = jnp.maximum(jnp.zeros((N, 1), jnp.float32).at[dst].add(1.0), 1.0)
    s1 = jnp.zeros((N, HEADS), jnp.float32).at[dst].add(e_raw)
    s2 = jnp.zeros((N, HEADS), jnp.float32).at[dst].add(e_raw * e_raw)
    mean = s1 / cnt
    var = jnp.maximum(s2 / cnt - mean * mean, 0.0)
    std = jnp.sqrt(var + 1e-6)
    e_norm = (e_raw - mean[dst]) / (std[dst] + 1e-6)
    e = jnp.tanh(e_norm)
    ex = jnp.exp(e)
    denom = jnp.zeros((N, HEADS), jnp.float32).at[dst].add(ex)
    alpha = ex / (denom[dst] + 1e-12)
    m = (alpha[:, :, None] * vi).reshape(E, OUT_DIM)
    out = jnp.zeros((N, OUT_DIM), jnp.float32).at[dst].add(m)
    return _final_proj(out, proj_W.T, proj_b[None, :])


# speedup vs baseline: 1.2816x; 1.1458x over previous
import functools
import math

import jax
import jax.numpy as jnp
import numpy as np
from jax import lax
from jax.experimental import pallas as pl
from jax.experimental.pallas import tpu as pltpu
from jax.experimental.pallas import tpu_sc as plsc

N = 10000
E = 160000
IN_DIM = 256
OUT_DIM = 256
HEADS = 8
DK = OUT_DIM // HEADS
LR_VOCAB = 512
LR_DIM = 32
EDGE_HID = 64
EDGE_BIAS = 32
FIJ_DIM = 6 + LR_DIM


def _qkv_body(x_ref, w_ref, out_ref):
    out_ref[...] = jnp.dot(x_ref[...], w_ref[...],
                           preferred_element_type=jnp.float32)


def _qkv(x, w_all):
    return pl.pallas_call(
        _qkv_body,
        out_shape=jax.ShapeDtypeStruct((N, 3 * OUT_DIM), jnp.float32),
        grid=(5,),
        in_specs=[
            pl.BlockSpec((N // 5, IN_DIM), lambda i: (i, 0)),
            pl.BlockSpec((IN_DIM, 3 * OUT_DIM), lambda i: (0, 0)),
        ],
        out_specs=pl.BlockSpec((N // 5, 3 * OUT_DIM), lambda i: (i, 0)),
    )(x, w_all)


EB = 8000


def _edge_body(cont_ref, lrid_ref, dot_ref, lrt_ref, w1_ref, b1_ref, w2_ref,
               b2_ref, wb_ref, eraw_ref):
    cont = cont_ref[...]
    lrid = lrid_ref[...]
    onehot = (lrid == jax.lax.broadcasted_iota(jnp.int32, (EB, LR_VOCAB), 1)
              ).astype(jnp.float32)
    lr_vec = jnp.dot(onehot, lrt_ref[...], preferred_element_type=jnp.float32)
    fij = jnp.concatenate([cont[:, :6], lr_vec], axis=1)
    h1 = jnp.maximum(jnp.dot(fij, w1_ref[...],
                             preferred_element_type=jnp.float32) + b1_ref[...], 0.0)
    phi = jnp.maximum(jnp.dot(h1, w2_ref[...],
                              preferred_element_type=jnp.float32) + b2_ref[...], 0.0)
    bias = jnp.dot(phi, wb_ref[...], preferred_element_type=jnp.float32)
    eraw_ref[...] = dot_ref[...] + bias


def _edge_bias(cont, lrid, dot, lr_table, phi_W1, phi_b1, phi_W2, phi_b2, wb):
    grid = (E // EB,)
    return pl.pallas_call(
        _edge_body,
        out_shape=jax.ShapeDtypeStruct((E, HEADS), jnp.float32),
        grid=grid,
        in_specs=[
            pl.BlockSpec((EB, 8), lambda i: (i, 0)),
            pl.BlockSpec((EB, 1), lambda i: (i, 0)),
            pl.BlockSpec((EB, HEADS), lambda i: (i, 0)),
            pl.BlockSpec((LR_VOCAB, LR_DIM), lambda i: (0, 0)),
            pl.BlockSpec((FIJ_DIM, EDGE_HID), lambda i: (0, 0)),
            pl.BlockSpec((1, EDGE_HID), lambda i: (0, 0)),
            pl.BlockSpec((EDGE_HID, EDGE_BIAS), lambda i: (0, 0)),
            pl.BlockSpec((1, EDGE_BIAS), lambda i: (0, 0)),
            pl.BlockSpec((EDGE_BIAS, HEADS), lambda i: (0, 0)),
        ],
        out_specs=pl.BlockSpec((EB, HEADS), lambda i: (i, 0)),
    )(cont, lrid, dot, lr_table, phi_W1, phi_b1, phi_W2, phi_b2, wb)


EPAD = 163840
EPT = EPAD // 16
CHUNK = 128
NCHUNK = EPT // CHUNK
NPAD = 10240
ROWS_PT = NPAD // 16


def _scatter_body(v2_hbm, src_hbm, dst_hbm, alpha_hbm, out_hbm, idx_v, idx2_v,
                  alpha_v, rows_v, acc_sh, sem):
    c = lax.axis_index("c")
    s = lax.axis_index("s")

    def _z(i, _):
        r = i // 8
        j = i - r * 8
        rows_v[r, pl.ds(j * 16, 16)] = jnp.zeros((16,), jnp.float32)
        return 0
    lax.fori_loop(0, CHUNK * 8, _z, 0)

    def _zcp(m, _):
        off = pl.multiple_of(s * ROWS_PT + m * CHUNK, CHUNK)
        pltpu.sync_copy(rows_v, acc_sh.at[pl.ds(off, CHUNK)])
        return 0
    lax.fori_loop(0, ROWS_PT // CHUNK, _zcp, 0)
    plsc.subcore_barrier()

    def _chunk(t, _):
        base = pl.multiple_of(s * EPT + t * CHUNK, CHUNK)
        pltpu.sync_copy(src_hbm.at[pl.ds(base, CHUNK)], idx2_v)
        pltpu.sync_copy(dst_hbm.at[pl.ds(base, CHUNK)], idx_v)
        pltpu.sync_copy(alpha_hbm.at[c, pl.ds(pl.multiple_of(base // 4, CHUNK // 4),
                                              CHUNK // 4)], alpha_v)

        def _mkidx(i, _):
            sl = pl.ds(i * 16, 16)
            idx2_v[sl] = idx2_v[sl] * 2 + c
            return 0
        lax.fori_loop(0, CHUNK // 16, _mkidx, 0)

        pltpu.async_copy(v2_hbm.at[idx2_v], rows_v, sem).wait()

        def _scale(g, _):
            av = alpha_v[g, :]
            for q in range(4):
                e = g * 4 + q
                for h in range(4):
                    a = jnp.full((16,), av[q * 4 + h])
                    for gg in range(2):
                        sl = pl.ds(h * 32 + gg * 16, 16)
                        rows_v[e, sl] = rows_v[e, sl] * a
            return 0
        lax.fori_loop(0, CHUNK // 4, _scale, 0)

        pltpu.sync_copy(rows_v, acc_sh.at[idx_v], add=True)
        return 0
    lax.fori_loop(0, NCHUNK, _chunk, 0)

    plsc.subcore_barrier()
    off = pl.multiple_of(s * ROWS_PT, ROWS_PT)
    pltpu.sync_copy(acc_sh.at[pl.ds(off, ROWS_PT)],
                    out_hbm.at[c, pl.ds(off, ROWS_PT)])


def _sc_scatter(v2, src_p, dst_p, alpha_p):
    mesh = plsc.VectorSubcoreMesh(core_axis_name="c", subcore_axis_name="s")
    return pl.kernel(
        _scatter_body,
        out_type=jax.ShapeDtypeStruct((2, NPAD, 128), jnp.float32),
        mesh=mesh,
        name="sc_alpha_scatter",
        scratch_types=[
            pltpu.VMEM((CHUNK,), jnp.int32),
            pltpu.VMEM((CHUNK,), jnp.int32),
            pltpu.VMEM((CHUNK // 4, 16), jnp.float32),
            pltpu.VMEM((CHUNK, 128), jnp.float32),
            pltpu.VMEM_SHARED((NPAD, 128), jnp.float32),
            pltpu.SemaphoreType.DMA,
        ],
    )(v2, src_p, dst_p, alpha_p)


def _proj_body(o0_ref, o1_ref, w0_ref, w1_ref, b_ref, res_ref):
    res_ref[...] = (jnp.dot(o0_ref[0], w0_ref[...],
                            preferred_element_type=jnp.float32)
                    + jnp.dot(o1_ref[0], w1_ref[...],
                              preferred_element_type=jnp.float32)
                    + b_ref[...])


def _final_proj(out2, proj_Wt, proj_b):
    return pl.pallas_call(
        _proj_body,
        out_shape=jax.ShapeDtypeStruct((N, OUT_DIM), jnp.float32),
        grid=(5,),
        in_specs=[
            pl.BlockSpec((1, N // 5, 128), lambda i: (0, i, 0)),
            pl.BlockSpec((1, N // 5, 128), lambda i: (1, i, 0)),
            pl.BlockSpec((128, OUT_DIM), lambda i: (0, 0)),
            pl.BlockSpec((128, OUT_DIM), lambda i: (0, 0)),
            pl.BlockSpec((1, OUT_DIM), lambda i: (0, 0)),
        ],
        out_specs=pl.BlockSpec((N // 5, OUT_DIM), lambda i: (i, 0)),
    )(out2, out2, proj_Wt[:128], proj_Wt[128:], proj_b)


def kernel(x, edge_index, edge_attr7, dist_sigma, coexpr_log1p, coexpr_scale,
           temperature, Wq, Wk, Wv, lr_table, phi_W1, phi_b1, phi_W2, phi_b2,
           wb, proj_W, proj_b):
    src = edge_index[0]
    dst = edge_index[1]

    w_all = jnp.concatenate([Wq.T, Wk.T, Wv.T], axis=1)
    qkv = _qkv(x, w_all)
    q = qkv[:, :OUT_DIM].reshape(N, HEADS, DK)
    k = qkv[:, OUT_DIM:2 * OUT_DIM].reshape(N, HEADS, DK)

    qj = q[dst]
    ki = k[src]
    tau = jnp.maximum(jnp.asarray(temperature, jnp.float32), 1e-6)
    dot = (qj * ki).sum(axis=-1) / np.sqrt(DK) / tau

    dist = edge_attr7[:, 0]
    coexpr = edge_attr7[:, 1]
    lr_id = edge_attr7[:, 2].astype(jnp.int32)
    cmi4 = edge_attr7[:, 3:7]
    sigma = jnp.maximum(jnp.asarray(dist_sigma, jnp.float32), 1e-6)
    dist_decay = jnp.exp(-dist / sigma)
    scale = jnp.maximum(jnp.asarray(coexpr_scale, jnp.float32), 1e-6)
    coexpr_norm = jnp.where(coexpr_log1p,
                            jnp.log1p(jnp.maximum(coexpr, 0.0)) / scale,
                            coexpr / scale)
    cmi4_norm = jnp.clip(cmi4, 0.0, 1.0)
    cont = jnp.concatenate([dist_decay[:, None], coexpr_norm[:, None],
                            cmi4_norm, jnp.zeros((E, 2), jnp.float32)], axis=1)

    e_raw = _edge_bias(cont, lr_id[:, None], dot, lr_table,
                       phi_W1.T, phi_b1[None, :], phi_W2.T, phi_b2[None, :],
                       wb.T)

    cnt = jnp.maximum(jnp.zeros((N, 1), jnp.float32).at[dst].add(1.0), 1.0)
    s1 = jnp.zeros((N, HEADS), jnp.float32).at[dst].add(e_raw)
    s2 = jnp.zeros((N, HEADS), jnp.float32).at[dst].add(e_raw * e_raw)
    mean = s1 / cnt
    var = jnp.maximum(s2 / cnt - mean * mean, 0.0)
    std = jnp.sqrt(var + 1e-6)
    e_norm = (e_raw - mean[dst]) / (std[dst] + 1e-6)
    e = jnp.tanh(e_norm)
    ex = jnp.exp(e)
    denom = jnp.zeros((N, HEADS), jnp.float32).at[dst].add(ex)
    alpha = ex / (denom[dst] + 1e-12)

    v2 = qkv[:, 2 * OUT_DIM:].reshape(2 * N, 128)
    src_p = jnp.concatenate([src, jnp.zeros(EPAD - E, jnp.int32)])
    dst_p = jnp.concatenate([dst, jnp.zeros(EPAD - E, jnp.int32)])
    alpha_t = jnp.transpose(alpha.reshape(E, 2, 4), (1, 0, 2))
    alpha_p = jnp.concatenate(
        [alpha_t, jnp.zeros((2, EPAD - E, 4), jnp.float32)], axis=1
    ).reshape(2, EPAD // 4, 16)
    out2 = _sc_scatter(v2, src_p, dst_p, alpha_p)[:, :N]
    return _final_proj(out2, proj_W.T, proj_b[None, :])


# speedup vs baseline: 1.4668x; 1.1445x over previous
import functools
import math

import jax
import jax.numpy as jnp
import numpy as np
from jax import lax
from jax.experimental import pallas as pl
from jax.experimental.pallas import tpu as pltpu
from jax.experimental.pallas import tpu_sc as plsc

N = 10000
E = 160000
IN_DIM = 256
OUT_DIM = 256
HEADS = 8
DK = OUT_DIM // HEADS
LR_VOCAB = 512
LR_DIM = 32
EDGE_HID = 64
EDGE_BIAS = 32
FIJ_DIM = 6 + LR_DIM


def _qkv_body(x_ref, w_ref, out_ref):
    out_ref[...] = jnp.dot(x_ref[...], w_ref[...],
                           preferred_element_type=jnp.float32)


def _qkv(x, w_all):
    return pl.pallas_call(
        _qkv_body,
        out_shape=jax.ShapeDtypeStruct((N, 3 * OUT_DIM), jnp.float32),
        grid=(5,),
        in_specs=[
            pl.BlockSpec((N // 5, IN_DIM), lambda i: (i, 0)),
            pl.BlockSpec((IN_DIM, 3 * OUT_DIM), lambda i: (0, 0)),
        ],
        out_specs=pl.BlockSpec((N // 5, 3 * OUT_DIM), lambda i: (i, 0)),
    )(x, w_all)


EB = 8000


def _edge_body(cont_ref, lrid_ref, dot_ref, lrt_ref, w1_ref, b1_ref, w2_ref,
               b2_ref, wb_ref, eraw_ref):
    cont = cont_ref[...]
    lrid = lrid_ref[...]
    onehot = (lrid == jax.lax.broadcasted_iota(jnp.int32, (EB, LR_VOCAB), 1)
              ).astype(jnp.float32)
    lr_vec = jnp.dot(onehot, lrt_ref[...], preferred_element_type=jnp.float32)
    fij = jnp.concatenate([cont[:, :6], lr_vec], axis=1)
    h1 = jnp.maximum(jnp.dot(fij, w1_ref[...],
                             preferred_element_type=jnp.float32) + b1_ref[...], 0.0)
    phi = jnp.maximum(jnp.dot(h1, w2_ref[...],
                              preferred_element_type=jnp.float32) + b2_ref[...], 0.0)
    bias = jnp.dot(phi, wb_ref[...], preferred_element_type=jnp.float32)
    eraw_ref[...] = dot_ref[...] + bias


def _edge_bias(cont, lrid, dot, lr_table, phi_W1, phi_b1, phi_W2, phi_b2, wb):
    grid = (E // EB,)
    return pl.pallas_call(
        _edge_body,
        out_shape=jax.ShapeDtypeStruct((E, HEADS), jnp.float32),
        grid=grid,
        in_specs=[
            pl.BlockSpec((EB, 8), lambda i: (i, 0)),
            pl.BlockSpec((EB, 1), lambda i: (i, 0)),
            pl.BlockSpec((EB, HEADS), lambda i: (i, 0)),
            pl.BlockSpec((LR_VOCAB, LR_DIM), lambda i: (0, 0)),
            pl.BlockSpec((FIJ_DIM, EDGE_HID), lambda i: (0, 0)),
            pl.BlockSpec((1, EDGE_HID), lambda i: (0, 0)),
            pl.BlockSpec((EDGE_HID, EDGE_BIAS), lambda i: (0, 0)),
            pl.BlockSpec((1, EDGE_BIAS), lambda i: (0, 0)),
            pl.BlockSpec((EDGE_BIAS, HEADS), lambda i: (0, 0)),
        ],
        out_specs=pl.BlockSpec((EB, HEADS), lambda i: (i, 0)),
    )(cont, lrid, dot, lr_table, phi_W1, phi_b1, phi_W2, phi_b2, wb)


EPAD = 163840
EPT = EPAD // 16
CHUNK = 128
NCHUNK = EPT // CHUNK
NPAD = 10240
ROWS_PT = NPAD // 16


EPSC = EPAD // 2
EPT_S = EPSC // 16
NCHUNK_S = EPT_S // CHUNK
SLANES = 32
SROWS_PT = NPAD // 16


def _stats_body(rows_hbm, dst_hbm, out_hbm, idx_v, rb_in, rb, acc_sh, sem):
    c = lax.axis_index("c")
    s = lax.axis_index("s")

    def _z(i, _):
        r = i // 8
        j = i - r * 8
        rb[r, pl.ds(j * 16, 16)] = jnp.zeros((16,), jnp.float32)
        return 0
    lax.fori_loop(0, CHUNK * 8, _z, 0)

    def _zcp(m, _):
        off = pl.multiple_of(s * SROWS_PT + m * CHUNK, CHUNK)
        pltpu.sync_copy(rb, acc_sh.at[pl.ds(off, CHUNK)])
        return 0
    lax.fori_loop(0, SROWS_PT // CHUNK, _zcp, 0)
    plsc.subcore_barrier()

    def _chunk(t, _):
        base = pl.multiple_of(c * EPSC + s * EPT_S + t * CHUNK, CHUNK)
        pltpu.sync_copy(dst_hbm.at[pl.ds(base, CHUNK)], idx_v)
        pltpu.sync_copy(rows_hbm.at[pl.ds(base, CHUNK)], rb_in)

        def _w(r, _):
            rb[r, pl.ds(0, 16)] = rb_in[r, pl.ds(0, 16)]
            rb[r, pl.ds(16, 16)] = rb_in[r, pl.ds(16, 16)]
            return 0
        lax.fori_loop(0, CHUNK, _w, 0)

        pltpu.sync_copy(rb, acc_sh.at[idx_v], add=True)
        return 0
    lax.fori_loop(0, NCHUNK_S, _chunk, 0)

    plsc.subcore_barrier()
    off = pl.multiple_of(s * SROWS_PT, SROWS_PT)
    pltpu.sync_copy(acc_sh.at[pl.ds(off, SROWS_PT)],
                    out_hbm.at[c, pl.ds(off, SROWS_PT)])


def _sc_stats(rows_p, dst_p):
    mesh = plsc.VectorSubcoreMesh(core_axis_name="c", subcore_axis_name="s")
    return pl.kernel(
        _stats_body,
        out_type=jax.ShapeDtypeStruct((2, NPAD, 128), jnp.float32),
        mesh=mesh,
        name="sc_stats",
        scratch_types=[
            pltpu.VMEM((CHUNK,), jnp.int32),
            pltpu.VMEM((CHUNK, SLANES), jnp.float32),
            pltpu.VMEM((CHUNK, 128), jnp.float32),
            pltpu.VMEM_SHARED((NPAD, 128), jnp.float32),
            pltpu.SemaphoreType.DMA,
        ],
    )(rows_p, dst_p)


VLANES = 128


def _scatter_body(v2_hbm, src_hbm, dst_hbm, ex_hbm, out_hbm, idx_v, idx2_v,
                  ex_v, rows_v, acc_sh, sem):
    c = lax.axis_index("c")
    s = lax.axis_index("s")

    def _z(i, _):
        r = i // 8
        j = i - r * 8
        rows_v[r, pl.ds(j * 16, 16)] = jnp.zeros((16,), jnp.float32)
        return 0
    lax.fori_loop(0, CHUNK * 8, _z, 0)

    def _zcp(m, _):
        off = pl.multiple_of(s * ROWS_PT + m * CHUNK, CHUNK)
        pltpu.sync_copy(rows_v, acc_sh.at[pl.ds(off, CHUNK)])
        return 0
    lax.fori_loop(0, ROWS_PT // CHUNK, _zcp, 0)
    plsc.subcore_barrier()

    def _chunk(t, _):
        base = pl.multiple_of(s * EPT + t * CHUNK, CHUNK)
        pltpu.sync_copy(src_hbm.at[pl.ds(base, CHUNK)], idx2_v)
        pltpu.sync_copy(dst_hbm.at[pl.ds(base, CHUNK)], idx_v)
        pltpu.sync_copy(ex_hbm.at[c, pl.ds(pl.multiple_of(base // 4, CHUNK // 4),
                                           CHUNK // 4)], ex_v)

        def _mkidx(i, _):
            sl = pl.ds(i * 16, 16)
            idx2_v[sl] = idx2_v[sl] * 2 + c
            return 0
        lax.fori_loop(0, CHUNK // 16, _mkidx, 0)

        pltpu.async_copy(v2_hbm.at[idx2_v], rows_v, sem).wait()

        def _scale(g, _):
            av = ex_v[g, :]
            for q in range(4):
                e = g * 4 + q
                for h in range(4):
                    a = jnp.full((16,), av[q * 4 + h])
                    for gg in range(2):
                        sl = pl.ds(h * 32 + gg * 16, 16)
                        rows_v[e, sl] = rows_v[e, sl] * a
            return 0
        lax.fori_loop(0, CHUNK // 4, _scale, 0)

        pltpu.sync_copy(rows_v, acc_sh.at[idx_v], add=True)
        return 0
    lax.fori_loop(0, NCHUNK, _chunk, 0)

    plsc.subcore_barrier()
    off = pl.multiple_of(s * ROWS_PT, ROWS_PT)
    pltpu.sync_copy(acc_sh.at[pl.ds(off, ROWS_PT)],
                    out_hbm.at[c, pl.ds(off, ROWS_PT)])


def _sc_scatter(v2, src_p, dst_p, ex_p):
    mesh = plsc.VectorSubcoreMesh(core_axis_name="c", subcore_axis_name="s")
    return pl.kernel(
        _scatter_body,
        out_type=jax.ShapeDtypeStruct((2, NPAD, VLANES), jnp.float32),
        mesh=mesh,
        name="sc_alpha_scatter",
        scratch_types=[
            pltpu.VMEM((CHUNK,), jnp.int32),
            pltpu.VMEM((CHUNK,), jnp.int32),
            pltpu.VMEM((CHUNK // 4, 16), jnp.float32),
            pltpu.VMEM((CHUNK, VLANES), jnp.float32),
            pltpu.VMEM_SHARED((NPAD, VLANES), jnp.float32),
            pltpu.SemaphoreType.DMA,
        ],
    )(v2, src_p, dst_p, ex_p)


def _proj_body(o0_ref, o1_ref, w0_ref, w1_ref, b_ref, res_ref):
    res_ref[...] = (jnp.dot(o0_ref[0], w0_ref[...],
                            preferred_element_type=jnp.float32)
                    + jnp.dot(o1_ref[0], w1_ref[...],
                              preferred_element_type=jnp.float32)
                    + b_ref[...])


def _final_proj(out2, proj_Wt, proj_b):
    return pl.pallas_call(
        _proj_body,
        out_shape=jax.ShapeDtypeStruct((N, OUT_DIM), jnp.float32),
        grid=(5,),
        in_specs=[
            pl.BlockSpec((1, N // 5, 128), lambda i: (0, i, 0)),
            pl.BlockSpec((1, N // 5, 128), lambda i: (1, i, 0)),
            pl.BlockSpec((128, OUT_DIM), lambda i: (0, 0)),
            pl.BlockSpec((128, OUT_DIM), lambda i: (0, 0)),
            pl.BlockSpec((1, OUT_DIM), lambda i: (0, 0)),
        ],
        out_specs=pl.BlockSpec((N // 5, OUT_DIM), lambda i: (i, 0)),
    )(out2, out2, proj_Wt[:128], proj_Wt[128:], proj_b)


def kernel(x, edge_index, edge_attr7, dist_sigma, coexpr_log1p, coexpr_scale,
           temperature, Wq, Wk, Wv, lr_table, phi_W1, phi_b1, phi_W2, phi_b2,
           wb, proj_W, proj_b):
    src = edge_index[0]
    dst = edge_index[1]

    w_all = jnp.concatenate([Wq.T, Wk.T, Wv.T], axis=1)
    qkv = _qkv(x, w_all)
    src_p = jnp.concatenate([src, jnp.zeros(EPAD - E, jnp.int32)])
    dst_p = jnp.concatenate([dst, jnp.full(EPAD - E, NPAD - 1, jnp.int32)])
    tau = jnp.maximum(jnp.asarray(temperature, jnp.float32), 1e-6)
    q = qkv[:, :OUT_DIM].reshape(N, HEADS, DK)
    k = qkv[:, OUT_DIM:2 * OUT_DIM].reshape(N, HEADS, DK)
    dot = (q[dst] * k[src]).sum(-1) / np.sqrt(DK) / tau

    dist = edge_attr7[:, 0]
    coexpr = edge_attr7[:, 1]
    lr_id = edge_attr7[:, 2].astype(jnp.int32)
    cmi4 = edge_attr7[:, 3:7]
    sigma = jnp.maximum(jnp.asarray(dist_sigma, jnp.float32), 1e-6)
    dist_decay = jnp.exp(-dist / sigma)
    scale = jnp.maximum(jnp.asarray(coexpr_scale, jnp.float32), 1e-6)
    coexpr_norm = jnp.where(coexpr_log1p,
                            jnp.log1p(jnp.maximum(coexpr, 0.0)) / scale,
                            coexpr / scale)
    cmi4_norm = jnp.clip(cmi4, 0.0, 1.0)
    cont = jnp.concatenate([dist_decay[:, None], coexpr_norm[:, None],
                            cmi4_norm, jnp.zeros((E, 2), jnp.float32)], axis=1)

    e_raw = _edge_bias(cont, lr_id[:, None], dot, lr_table,
                       phi_W1.T, phi_b1[None, :], phi_W2.T, phi_b2[None, :],
                       wb.T)

    srows = jnp.concatenate([e_raw, e_raw * e_raw,
                             jnp.ones((E, 1), jnp.float32),
                             jnp.zeros((E, SLANES - 2 * HEADS - 1), jnp.float32)],
                            axis=1)
    srows_p = jnp.concatenate([srows, jnp.zeros((EPAD - E, SLANES), jnp.float32)])
    stats2 = _sc_stats(srows_p, dst_p)
    stats = stats2[0, :N, :2 * HEADS + 1] + stats2[1, :N, :2 * HEADS + 1]
    s1 = stats[:, :HEADS]
    s2 = stats[:, HEADS:2 * HEADS]
    cnt = jnp.maximum(stats[:, 2 * HEADS:2 * HEADS + 1], 1.0)
    mean = s1 / cnt
    var = jnp.maximum(s2 / cnt - mean * mean, 0.0)
    std = jnp.sqrt(var + 1e-6)
    e_norm = (e_raw - mean[dst]) / (std[dst] + 1e-6)
    e = jnp.tanh(e_norm)
    ex = jnp.exp(e)
    denom = jnp.zeros((N, HEADS), jnp.float32).at[dst].add(ex)
    alpha = ex / (denom[dst] + 1e-12)

    v2 = qkv[:, 2 * OUT_DIM:].reshape(2 * N, 128)
    alpha_t = jnp.transpose(alpha.reshape(E, 2, 4), (1, 0, 2))
    alpha_p = jnp.concatenate(
        [alpha_t, jnp.zeros((2, EPAD - E, 4), jnp.float32)], axis=1
    ).reshape(2, EPAD // 4, 16)
    out2 = _sc_scatter(v2, src_p, dst_p, alpha_p)
    return _final_proj(out2, proj_W.T, proj_b[None, :])


# speedup vs baseline: 1.7928x; 1.2223x over previous
import functools
import math

import jax
import jax.numpy as jnp
import numpy as np
from jax import lax
from jax.experimental import pallas as pl
from jax.experimental.pallas import tpu as pltpu
from jax.experimental.pallas import tpu_sc as plsc

N = 10000
E = 160000
IN_DIM = 256
OUT_DIM = 256
HEADS = 8
DK = OUT_DIM // HEADS
LR_VOCAB = 512
LR_DIM = 32
EDGE_HID = 64
EDGE_BIAS = 32
FIJ_DIM = 6 + LR_DIM


def _qkv_body(x_ref, w_ref, out_ref):
    out_ref[...] = jnp.dot(x_ref[...], w_ref[...],
                           preferred_element_type=jnp.float32)


def _qkv(x, w_all):
    return pl.pallas_call(
        _qkv_body,
        out_shape=jax.ShapeDtypeStruct((N, 3 * OUT_DIM), jnp.float32),
        grid=(5,),
        in_specs=[
            pl.BlockSpec((N // 5, IN_DIM), lambda i: (i, 0)),
            pl.BlockSpec((IN_DIM, 3 * OUT_DIM), lambda i: (0, 0)),
        ],
        out_specs=pl.BlockSpec((N // 5, 3 * OUT_DIM), lambda i: (i, 0)),
    )(x, w_all)


EB = 8000


def _edge_body(cont_ref, lrid_ref, dot_ref, lrt_ref, w1_ref, b1_ref, w2_ref,
               b2_ref, wb_ref, eraw_ref):
    cont = cont_ref[...]
    lrid = lrid_ref[...]
    onehot = (lrid == jax.lax.broadcasted_iota(jnp.int32, (EB, LR_VOCAB), 1)
              ).astype(jnp.float32)
    lr_vec = jnp.dot(onehot, lrt_ref[...], preferred_element_type=jnp.float32)
    fij = jnp.concatenate([cont[:, :6], lr_vec], axis=1)
    h1 = jnp.maximum(jnp.dot(fij, w1_ref[...],
                             preferred_element_type=jnp.float32) + b1_ref[...], 0.0)
    phi = jnp.maximum(jnp.dot(h1, w2_ref[...],
                              preferred_element_type=jnp.float32) + b2_ref[...], 0.0)
    bias = jnp.dot(phi, wb_ref[...], preferred_element_type=jnp.float32)
    eraw_ref[...] = dot_ref[...] + bias


def _edge_bias(cont, lrid, dot, lr_table, phi_W1, phi_b1, phi_W2, phi_b2, wb):
    grid = (E // EB,)
    return pl.pallas_call(
        _edge_body,
        out_shape=jax.ShapeDtypeStruct((E, HEADS), jnp.float32),
        grid=grid,
        in_specs=[
            pl.BlockSpec((EB, 8), lambda i: (i, 0)),
            pl.BlockSpec((EB, 1), lambda i: (i, 0)),
            pl.BlockSpec((EB, HEADS), lambda i: (i, 0)),
            pl.BlockSpec((LR_VOCAB, LR_DIM), lambda i: (0, 0)),
            pl.BlockSpec((FIJ_DIM, EDGE_HID), lambda i: (0, 0)),
            pl.BlockSpec((1, EDGE_HID), lambda i: (0, 0)),
            pl.BlockSpec((EDGE_HID, EDGE_BIAS), lambda i: (0, 0)),
            pl.BlockSpec((1, EDGE_BIAS), lambda i: (0, 0)),
            pl.BlockSpec((EDGE_BIAS, HEADS), lambda i: (0, 0)),
        ],
        out_specs=pl.BlockSpec((EB, HEADS), lambda i: (i, 0)),
    )(cont, lrid, dot, lr_table, phi_W1, phi_b1, phi_W2, phi_b2, wb)


EPAD = 163840
EPT = EPAD // 16
CHUNK = 128
NCHUNK = EPT // CHUNK
NPAD = 10240
ROWS_PT = NPAD // 16


EPSC_D = EPAD // 2
EPT_D = EPSC_D // 16
NCHUNK_D = EPT_D // CHUNK


def _dot_body(qkv_hbm, src_hbm, dst_hbm, out_hbm, idxq_v, idxk_v, qb, kb,
              sb, sem, sem2):
    c = lax.axis_index("c")
    s = lax.axis_index("s")
    iota = lax.iota(jnp.int32, 16)
    masks = [iota == h for h in range(HEADS)]

    def _chunk(t, _):
        base = pl.multiple_of(c * EPSC_D + s * EPT_D + t * CHUNK, CHUNK)
        pltpu.sync_copy(dst_hbm.at[pl.ds(base, CHUNK)], idxq_v)
        pltpu.sync_copy(src_hbm.at[pl.ds(base, CHUNK)], idxk_v)

        def _mkidx(i, _):
            sl = pl.ds(i * 16, 16)
            idxq_v[sl] = idxq_v[sl] * 3
            idxk_v[sl] = idxk_v[sl] * 3 + 1
            return 0
        lax.fori_loop(0, CHUNK // 16, _mkidx, 0)

        cp1 = pltpu.async_copy(qkv_hbm.at[idxq_v], qb, sem)
        cp2 = pltpu.async_copy(qkv_hbm.at[idxk_v], kb, sem2)
        cp1.wait()
        cp2.wait()

        def _edge(e, _):
            dv = jnp.zeros((16,), jnp.float32)
            for h in range(HEADS):
                sl0 = pl.ds(h * 32, 16)
                sl1 = pl.ds(h * 32 + 16, 16)
                m = qb[e, sl0] * kb[e, sl0] + qb[e, sl1] * kb[e, sl1]
                dv = jnp.where(masks[h], jnp.full((16,), jnp.sum(m)), dv)
            sb[e, :] = dv
            return 0
        lax.fori_loop(0, CHUNK, _edge, 0)

        pltpu.sync_copy(sb, out_hbm.at[pl.ds(base, CHUNK)])
        return 0
    lax.fori_loop(0, NCHUNK_D, _chunk, 0)


def _sc_dot(qkv3, src_p, dst_p):
    mesh = plsc.VectorSubcoreMesh(core_axis_name="c", subcore_axis_name="s")
    return pl.kernel(
        _dot_body,
        out_type=jax.ShapeDtypeStruct((EPAD, 16), jnp.float32),
        mesh=mesh,
        name="sc_dot",
        scratch_types=[
            pltpu.VMEM((CHUNK,), jnp.int32),
            pltpu.VMEM((CHUNK,), jnp.int32),
            pltpu.VMEM((CHUNK, OUT_DIM), jnp.float32),
            pltpu.VMEM((CHUNK, OUT_DIM), jnp.float32),
            pltpu.VMEM((CHUNK, 16), jnp.float32),
            pltpu.SemaphoreType.DMA,
            pltpu.SemaphoreType.DMA,
        ],
    )(qkv3, src_p, dst_p)


EPSC = EPAD // 2
EPT_S = EPSC // 16
NCHUNK_S = EPT_S // CHUNK
SLANES = 32
SROWS_PT = NPAD // 16


def _stats_body(rows_hbm, dst_hbm, out_hbm, idx_v, rb_in, rb, acc_sh, sem):
    c = lax.axis_index("c")
    s = lax.axis_index("s")

    def _z(i, _):
        r = i // 8
        j = i - r * 8
        rb[r, pl.ds(j * 16, 16)] = jnp.zeros((16,), jnp.float32)
        return 0
    lax.fori_loop(0, CHUNK * 8, _z, 0)

    def _zcp(m, _):
        off = pl.multiple_of(s * SROWS_PT + m * CHUNK, CHUNK)
        pltpu.sync_copy(rb, acc_sh.at[pl.ds(off, CHUNK)])
        return 0
    lax.fori_loop(0, SROWS_PT // CHUNK, _zcp, 0)
    plsc.subcore_barrier()

    def _chunk(t, _):
        base = pl.multiple_of(c * EPSC + s * EPT_S + t * CHUNK, CHUNK)
        pltpu.sync_copy(dst_hbm.at[pl.ds(base, CHUNK)], idx_v)
        pltpu.sync_copy(rows_hbm.at[pl.ds(base, CHUNK)], rb_in)

        def _w(r, _):
            rb[r, pl.ds(0, 16)] = rb_in[r, pl.ds(0, 16)]
            rb[r, pl.ds(16, 16)] = rb_in[r, pl.ds(16, 16)]
            return 0
        lax.fori_loop(0, CHUNK, _w, 0)

        pltpu.sync_copy(rb, acc_sh.at[idx_v], add=True)
        return 0
    lax.fori_loop(0, NCHUNK_S, _chunk, 0)

    plsc.subcore_barrier()
    off = pl.multiple_of(s * SROWS_PT, SROWS_PT)
    pltpu.sync_copy(acc_sh.at[pl.ds(off, SROWS_PT)],
                    out_hbm.at[c, pl.ds(off, SROWS_PT)])


def _sc_stats(rows_p, dst_p):
    mesh = plsc.VectorSubcoreMesh(core_axis_name="c", subcore_axis_name="s")
    return pl.kernel(
        _stats_body,
        out_type=jax.ShapeDtypeStruct((2, NPAD, 128), jnp.float32),
        mesh=mesh,
        name="sc_stats",
        scratch_types=[
            pltpu.VMEM((CHUNK,), jnp.int32),
            pltpu.VMEM((CHUNK, SLANES), jnp.float32),
            pltpu.VMEM((CHUNK, 128), jnp.float32),
            pltpu.VMEM_SHARED((NPAD, 128), jnp.float32),
            pltpu.SemaphoreType.DMA,
        ],
    )(rows_p, dst_p)


VLANES = 128


def _scatter_body(v2_hbm, src_hbm, dst_hbm, ex_hbm, out_hbm, idx_v, idx2_v,
                  ex_v, rows_v, acc_sh, sem):
    c = lax.axis_index("c")
    s = lax.axis_index("s")

    def _z(i, _):
        r = i // 8
        j = i - r * 8
        rows_v[r, pl.ds(j * 16, 16)] = jnp.zeros((16,), jnp.float32)
        return 0
    lax.fori_loop(0, CHUNK * 8, _z, 0)

    def _zcp(m, _):
        off = pl.multiple_of(s * ROWS_PT + m * CHUNK, CHUNK)
        pltpu.sync_copy(rows_v, acc_sh.at[pl.ds(off, CHUNK)])
        return 0
    lax.fori_loop(0, ROWS_PT // CHUNK, _zcp, 0)
    plsc.subcore_barrier()

    def _chunk(t, _):
        base = pl.multiple_of(s * EPT + t * CHUNK, CHUNK)
        pltpu.sync_copy(src_hbm.at[pl.ds(base, CHUNK)], idx2_v)
        pltpu.sync_copy(dst_hbm.at[pl.ds(base, CHUNK)], idx_v)
        pltpu.sync_copy(ex_hbm.at[c, pl.ds(pl.multiple_of(base // 4, CHUNK // 4),
                                           CHUNK // 4)], ex_v)

        def _mkidx(i, _):
            sl = pl.ds(i * 16, 16)
            idx2_v[sl] = idx2_v[sl] * 2 + c
            return 0
        lax.fori_loop(0, CHUNK // 16, _mkidx, 0)

        pltpu.async_copy(v2_hbm.at[idx2_v], rows_v, sem).wait()

        def _scale(g, _):
            av = ex_v[g, :]
            for q in range(4):
                e = g * 4 + q
                for h in range(4):
                    a = jnp.full((16,), av[q * 4 + h])
                    for gg in range(2):
                        sl = pl.ds(h * 32 + gg * 16, 16)
                        rows_v[e, sl] = rows_v[e, sl] * a
            return 0
        lax.fori_loop(0, CHUNK // 4, _scale, 0)

        pltpu.sync_copy(rows_v, acc_sh.at[idx_v], add=True)
        return 0
    lax.fori_loop(0, NCHUNK, _chunk, 0)

    plsc.subcore_barrier()
    off = pl.multiple_of(s * ROWS_PT, ROWS_PT)
    pltpu.sync_copy(acc_sh.at[pl.ds(off, ROWS_PT)],
                    out_hbm.at[c, pl.ds(off, ROWS_PT)])


def _sc_scatter(v2, src_p, dst_p, ex_p):
    mesh = plsc.VectorSubcoreMesh(core_axis_name="c", subcore_axis_name="s")
    return pl.kernel(
        _scatter_body,
        out_type=jax.ShapeDtypeStruct((2, NPAD, VLANES), jnp.float32),
        mesh=mesh,
        name="sc_alpha_scatter",
        scratch_types=[
            pltpu.VMEM((CHUNK,), jnp.int32),
            pltpu.VMEM((CHUNK,), jnp.int32),
            pltpu.VMEM((CHUNK // 4, 16), jnp.float32),
            pltpu.VMEM((CHUNK, VLANES), jnp.float32),
            pltpu.VMEM_SHARED((NPAD, VLANES), jnp.float32),
            pltpu.SemaphoreType.DMA,
        ],
    )(v2, src_p, dst_p, ex_p)


def _proj_body(o0_ref, o1_ref, w0_ref, w1_ref, b_ref, res_ref):
    res_ref[...] = (jnp.dot(o0_ref[0], w0_ref[...],
                            preferred_element_type=jnp.float32)
                    + jnp.dot(o1_ref[0], w1_ref[...],
                              preferred_element_type=jnp.float32)
                    + b_ref[...])


def _final_proj(out2, proj_Wt, proj_b):
    return pl.pallas_call(
        _proj_body,
        out_shape=jax.ShapeDtypeStruct((N, OUT_DIM), jnp.float32),
        grid=(5,),
        in_specs=[
            pl.BlockSpec((1, N // 5, 128), lambda i: (0, i, 0)),
            pl.BlockSpec((1, N // 5, 128), lambda i: (1, i, 0)),
            pl.BlockSpec((128, OUT_DIM), lambda i: (0, 0)),
            pl.BlockSpec((128, OUT_DIM), lambda i: (0, 0)),
            pl.BlockSpec((1, OUT_DIM), lambda i: (0, 0)),
        ],
        out_specs=pl.BlockSpec((N // 5, OUT_DIM), lambda i: (i, 0)),
    )(out2, out2, proj_Wt[:128], proj_Wt[128:], proj_b)


def kernel(x, edge_index, edge_attr7, dist_sigma, coexpr_log1p, coexpr_scale,
           temperature, Wq, Wk, Wv, lr_table, phi_W1, phi_b1, phi_W2, phi_b2,
           wb, proj_W, proj_b):
    src = edge_index[0]
    dst = edge_index[1]

    w_all = jnp.concatenate([Wq.T, Wk.T, Wv.T], axis=1)
    qkv = _qkv(x, w_all)
    src_p = jnp.concatenate([src, jnp.zeros(EPAD - E, jnp.int32)])
    dst_p = jnp.concatenate([dst, jnp.full(EPAD - E, NPAD - 1, jnp.int32)])
    tau = jnp.maximum(jnp.asarray(temperature, jnp.float32), 1e-6)
    q = qkv[:, :OUT_DIM].reshape(N, HEADS, DK)
    k = qkv[:, OUT_DIM:2 * OUT_DIM].reshape(N, HEADS, DK)
    qkv3 = qkv.reshape(3 * N, OUT_DIM)
    qd = jnp.take(qkv3, dst * 3, axis=0)
    kd = jnp.take(qkv3, src * 3 + 1, axis=0)
    dot = (qd * kd).reshape(E, HEADS, DK).sum(-1) / np.sqrt(DK) / tau

    dist = edge_attr7[:, 0]
    coexpr = edge_attr7[:, 1]
    lr_id = edge_attr7[:, 2].astype(jnp.int32)
    cmi4 = edge_attr7[:, 3:7]
    sigma = jnp.maximum(jnp.asarray(dist_sigma, jnp.float32), 1e-6)
    dist_decay = jnp.exp(-dist / sigma)
    scale = jnp.maximum(jnp.asarray(coexpr_scale, jnp.float32), 1e-6)
    coexpr_norm = jnp.where(coexpr_log1p,
                            jnp.log1p(jnp.maximum(coexpr, 0.0)) / scale,
                            coexpr / scale)
    cmi4_norm = jnp.clip(cmi4, 0.0, 1.0)
    cont = jnp.concatenate([dist_decay[:, None], coexpr_norm[:, None],
                            cmi4_norm, jnp.zeros((E, 2), jnp.float32)], axis=1)

    e_raw = _edge_bias(cont, lr_id[:, None], dot, lr_table,
                       phi_W1.T, phi_b1[None, :], phi_W2.T, phi_b2[None, :],
                       wb.T)

    srows = jnp.concatenate([e_raw, e_raw * e_raw,
                             jnp.ones((E, 1), jnp.float32),
                             jnp.zeros((E, SLANES - 2 * HEADS - 1), jnp.float32)],
                            axis=1)
    srows_p = jnp.concatenate([srows, jnp.zeros((EPAD - E, SLANES), jnp.float32)])
    stats2 = _sc_stats(srows_p, dst_p)
    stats = stats2[0, :N, :2 * HEADS + 1] + stats2[1, :N, :2 * HEADS + 1]
    s1 = stats[:, :HEADS]
    s2 = stats[:, HEADS:2 * HEADS]
    cnt = jnp.maximum(stats[:, 2 * HEADS:2 * HEADS + 1], 1.0)
    mean = s1 / cnt
    var = jnp.maximum(s2 / cnt - mean * mean, 0.0)
    std = jnp.sqrt(var + 1e-6)
    e_norm = (e_raw - mean[dst]) / (std[dst] + 1e-6)
    e = jnp.tanh(e_norm)
    ex = jnp.exp(e)
    denom = jnp.zeros((N, HEADS), jnp.float32).at[dst].add(ex)
    alpha = ex / (denom[dst] + 1e-12)

    v2 = qkv[:, 2 * OUT_DIM:].reshape(2 * N, 128)
    alpha_t = jnp.transpose(alpha.reshape(E, 2, 4), (1, 0, 2))
    alpha_p = jnp.concatenate(
        [alpha_t, jnp.zeros((2, EPAD - E, 4), jnp.float32)], axis=1
    ).reshape(2, EPAD // 4, 16)
    out2 = _sc_scatter(v2, src_p, dst_p, alpha_p)
    return _final_proj(out2, proj_W.T, proj_b[None, :])
